# baseline (speedup 1.0000x reference)
import jax
import jax.numpy as jnp
from jax import lax
from jax.experimental import pallas as pl
from jax.experimental.pallas import tpu as pltpu
from jax.experimental.pallas import tpu_sc as plsc

_NC = 2
_NS = 16
_LANES = 16


def _scores_argmin_kernel(batch_ref, w_ref, bmu_ref):
    b = batch_ref[...]
    w = w_ref[...]
    bw = lax.dot_general(b, w, (((1,), (1,)), ((), ())),
                         preferred_element_type=jnp.float32,
                         precision=lax.Precision.HIGHEST)
    ones_row = jnp.ones((1, b.shape[1]), dtype=jnp.float32)
    wn = lax.dot_general(ones_row, w * w, (((1,), (1,)), ((), ())),
                         preferred_element_type=jnp.float32,
                         precision=lax.Precision.HIGHEST)
    scores = wn - 2.0 * bw
    row_min = jnp.min(scores, axis=1, keepdims=True)
    col = lax.broadcasted_iota(jnp.int32, scores.shape, 1)
    m_total = scores.shape[1]
    bmu_ref[...] = jnp.min(jnp.where(scores <= row_min, col, m_total),
                           axis=1, keepdims=True)


def _sc_gather_body(bmu_hbm, locT_hbm, out_hbm, idx_v, loci_v, locj_v, out_v):
    wid = lax.axis_index("s") * _NC + lax.axis_index("c")

    @pl.when(wid < 16)
    def _():
        base = wid * _LANES
        pltpu.sync_copy(bmu_hbm.at[pl.ds(base, _LANES)], idx_v)
        pltpu.sync_copy(locT_hbm.at[0], loci_v)
        pltpu.sync_copy(locT_hbm.at[1], locj_v)
        idx = idx_v[...]
        li = plsc.load_gather(loci_v, [idx])
        lj = plsc.load_gather(locj_v, [idx])
        lane = lax.iota(jnp.int32, _LANES)
        plsc.store_scatter(out_v, [lane * 2], li)
        plsc.store_scatter(out_v, [lane * 2 + 1], lj)
        pltpu.sync_copy(out_v, out_hbm.at[pl.ds(base * 2, 2 * _LANES)])


def _sc_gather(bmu_flat, locT):
    B = bmu_flat.shape[0]
    M = locT.shape[1]
    mesh = plsc.VectorSubcoreMesh(core_axis_name="c", subcore_axis_name="s")
    return pl.kernel(
        _sc_gather_body,
        out_type=jax.ShapeDtypeStruct((2 * B,), jnp.float32),
        mesh=mesh,
        compiler_params=pltpu.CompilerParams(needs_layout_passes=False),
        scratch_types=[
            pltpu.VMEM((_LANES,), jnp.int32),
            pltpu.VMEM((M,), jnp.float32),
            pltpu.VMEM((M,), jnp.float32),
            pltpu.VMEM((2 * _LANES,), jnp.float32),
        ],
    )(bmu_flat, locT)


def _neigh_kernel(locT_ref, bloc_ref, invr2_ref, out_ref):
    locT = locT_ref[...]
    loc_i = locT[0:1, :]
    loc_j = locT[1:2, :]
    bi = bloc_ref[:, 0:1]
    bj = bloc_ref[:, 1:2]
    d2 = (loc_i - bi) ** 2 + (loc_j - bj) ** 2
    out_ref[...] = jnp.exp(-(d2 * invr2_ref[0, 0]))


def kernel(batch, weights, locations, radius):
    B = batch.shape[0]
    M = weights.shape[0]
    locT = locations.astype(jnp.float32).T
    inv_r2 = (1.0 / (jnp.asarray(radius).astype(jnp.float32) ** 2)
              ).reshape(1, 1)

    bmu = pl.pallas_call(
        _scores_argmin_kernel,
        out_shape=jax.ShapeDtypeStruct((B, 1), jnp.int32),
    )(batch, weights)

    bloc = _sc_gather(bmu.reshape(-1), locT).reshape(B, 2)

    return pl.pallas_call(
        _neigh_kernel,
        out_shape=jax.ShapeDtypeStruct((B, M), jnp.float32),
    )(locT, bloc, inv_r2)
